# baseline (speedup 1.0000x reference)
import functools

import jax
import jax.numpy as jnp
import numpy as np
from jax import lax
from jax.experimental import pallas as pl
from jax.experimental.pallas import tpu as pltpu
from jax.experimental.pallas import tpu_sc as plsc

_LANES = 16


def _make_sc_pool(B, L, V, D, num_cores, num_subcores):
    NW = num_cores * num_subcores
    assert B % NW == 0
    b_per_w = B // NW
    assert b_per_w % 2 == 0
    D32 = D // 32
    assert D % 32 == 0
    CH = 40
    assert L % CH == 0 and (CH * 4) % 8 == 0
    n_chunks = L // CH
    n_tot = b_per_w * n_chunks
    RU = 4
    assert CH % RU == 0

    mesh = plsc.VectorSubcoreMesh(core_axis_name="c", subcore_axis_name="s")

    @functools.partial(
        pl.kernel,
        mesh=mesh,
        out_type=jax.ShapeDtypeStruct((B, D), jnp.float32),
        scratch_types=[
            pltpu.VMEM((b_per_w * L,), jnp.int32),
            pltpu.VMEM((2, CH, D // 2), jnp.int32),
            pltpu.VMEM((2, D), jnp.float32),
            pltpu.SemaphoreType.DMA((2,)),
            pltpu.SemaphoreType.DMA((2,)),
        ],
        compiler_params=pltpu.CompilerParams(needs_layout_passes=False),
    )
    def sc_pool(tok_hbm, table_hbm, out_hbm, tok_v, rows_v, stage_v, gsem, osem):
        wid = lax.axis_index("s") * num_cores + lax.axis_index("c")
        base = wid * b_per_w
        pltpu.sync_copy(tok_hbm.at[pl.ds(base * L, b_per_w * L)], tok_v)

        def gather(k, s):
            return pltpu.make_async_copy(
                table_hbm.at[tok_v.at[pl.ds(k * CH, CH)]],
                rows_v.at[s],
                gsem.at[s],
            )

        gather(0, 0).start()

        def chunk_body(k, carry):
            accs = carry
            s = lax.rem(k, 2)
            gather(k, s).wait()

            @pl.when(k + 1 < n_tot)
            def _():
                gather(k + 1, 1 - s).start()

            def row_body(r4, accs):
                accs = list(accs)
                for u in range(RU):
                    r = r4 * RU + u
                    for j in range(D32):
                        xi = rows_v[s, r, pl.ds(j * 16, 16)]
                        f_hi = plsc.bitcast(xi, jnp.float32)
                        f_lo = plsc.bitcast(
                            lax.shift_left(xi, 16), jnp.float32
                        )
                        accs[2 * j] = accs[2 * j] + f_lo
                        accs[2 * j + 1] = accs[2 * j + 1] + f_hi
                return tuple(accs)

            accs = lax.fori_loop(0, CH // RU, row_body, accs)

            c = lax.rem(k, n_chunks)
            i = lax.div(k, n_chunks)
            q = lax.rem(i, 2)

            @pl.when(c == n_chunks - 1)
            def _():
                @pl.when(i >= 2)
                def _():
                    pltpu.make_async_copy(
                        stage_v.at[q], out_hbm.at[base + i - 2], osem.at[q]
                    ).wait()
                for j in range(D32):
                    stage_v[q, pl.ds(j * 32, _LANES)] = accs[2 * j]
                    stage_v[q, pl.ds(j * 32 + _LANES, _LANES)] = accs[2 * j + 1]
                pltpu.make_async_copy(
                    stage_v.at[q], out_hbm.at[base + i], osem.at[q]
                ).start()

            zero = jnp.zeros((_LANES,), jnp.float32)
            return tuple(
                jnp.where(c == n_chunks - 1, zero, a) for a in accs
            )

        zeros = tuple(jnp.zeros((_LANES,), jnp.float32) for _ in range(2 * D32))
        lax.fori_loop(0, n_tot, chunk_body, zeros)

        pltpu.make_async_copy(
            stage_v.at[0], out_hbm.at[base + b_per_w - 2], osem.at[0]
        ).wait()
        pltpu.make_async_copy(
            stage_v.at[1], out_hbm.at[base + b_per_w - 1], osem.at[1]
        ).wait()

    return sc_pool


def _pool_perm(D):
    p = np.empty(D, np.int32)
    t = 2 * np.arange(16, dtype=np.int32)
    for j in range(D // 32):
        p[32 * j : 32 * j + 16] = 32 * j + t
        p[32 * j + 16 : 32 * j + 32] = 32 * j + t + 1
    return p


def _tc_proj_body(x_ref, w_ref, b_ref, o_ref, *, inv_l):
    x = x_ref[...] * inv_l
    y = lax.dot_general(
        x, w_ref[...], (((1,), (1,)), ((), ())),
        preferred_element_type=jnp.float32,
    )
    y = y + b_ref[...]
    norm = jnp.sqrt(jnp.sum(y * y, axis=-1, keepdims=True))
    o_ref[...] = y / jnp.maximum(norm, 1e-12)


def _tc_proj(x, W, b2d, L):
    B, D = x.shape
    BB = 256
    assert B % BB == 0
    return pl.pallas_call(
        functools.partial(_tc_proj_body, inv_l=1.0 / L),
        grid=(B // BB,),
        in_specs=[
            pl.BlockSpec((BB, D), lambda i: (i, 0)),
            pl.BlockSpec((D, D), lambda i: (0, 0)),
            pl.BlockSpec((1, D), lambda i: (0, 0)),
        ],
        out_specs=pl.BlockSpec((BB, D), lambda i: (i, 0)),
        out_shape=jax.ShapeDtypeStruct((B, D), jnp.float32),
    )(x, W, b2d)


def kernel(tokens, emb, W, b):
    B, L = tokens.shape
    V, D = emb.shape
    info = plsc.get_sparse_core_info()
    sc_pool = _make_sc_pool(B, L, V, D, info.num_cores, info.num_subcores)
    emb_packed = lax.bitcast_convert_type(
        emb.astype(jnp.bfloat16).reshape(V, D // 2, 2), jnp.int32
    )
    pooled = sc_pool(tokens.astype(jnp.int32).reshape(B * L), emb_packed)
    W_used = W[:, _pool_perm(D)]
    return _tc_proj(pooled, W_used, b.reshape(1, D), L)


# speedup vs baseline: 1.1269x; 1.1269x over previous
import functools

import jax
import jax.numpy as jnp
from jax import lax
from jax.experimental import pallas as pl
from jax.experimental.pallas import tpu as pltpu
from jax.experimental.pallas import tpu_sc as plsc

_LANES = 16


def _make_sc_pack(V, D, num_cores, num_subcores):
    NW = num_cores * num_subcores
    PC = 32
    n_chunks_tot = -(-V // PC)
    cpw = -(-n_chunks_tot // NW)
    D2 = D // 2
    DV = D // 32

    mesh = plsc.VectorSubcoreMesh(core_axis_name="c", subcore_axis_name="s")

    @functools.partial(
        pl.kernel,
        mesh=mesh,
        out_type=jax.ShapeDtypeStruct((V * D2,), jnp.int32),
        scratch_types=[
            pltpu.VMEM((2, PC * D), jnp.int32),
            pltpu.VMEM((2, PC * D2), jnp.int32),
            pltpu.SemaphoreType.DMA((2,)),
            pltpu.SemaphoreType.DMA((2,)),
        ],
        compiler_params=pltpu.CompilerParams(needs_layout_passes=False),
    )
    def sc_pack(tab_hbm, out_hbm, in_v, out_v, isem, osem):
        wid = lax.axis_index("s") * num_cores + lax.axis_index("c")

        def row_start(c):
            g = wid * cpw + c
            return jnp.minimum(g * PC, V - PC)

        def copy_in(c, s):
            return pltpu.make_async_copy(
                tab_hbm.at[pl.ds(row_start(c) * D, PC * D)],
                in_v.at[s],
                isem.at[s],
            )

        def copy_out(c, s):
            return pltpu.make_async_copy(
                out_v.at[s],
                out_hbm.at[pl.ds(row_start(c) * D2, PC * D2)],
                osem.at[s],
            )

        n_mine = jnp.minimum(cpw, n_chunks_tot - wid * cpw)

        @pl.when(n_mine > 0)
        def _():
            copy_in(0, 0).start()

        def chunk_body(c, carry):
            s = lax.rem(c, 2)
            copy_in(c, s).wait()

            @pl.when(c + 1 < n_mine)
            def _():
                copy_in(c + 1, 1 - s).start()

            @pl.when(c >= 2)
            def _():
                copy_out(c - 2, s).wait()

            himask = jnp.full((_LANES,), -65536, jnp.int32)

            def row_body(r, carry):
                for j in range(DV):
                    a = in_v[s, pl.ds(r * D + 32 * j, _LANES)]
                    b = in_v[s, pl.ds(r * D + 32 * j + _LANES, _LANES)]
                    packed = jnp.bitwise_or(
                        lax.shift_right_logical(a, 16),
                        jnp.bitwise_and(b, himask),
                    )
                    out_v[s, pl.ds(r * D2 + _LANES * j, _LANES)] = packed
                return carry

            lax.fori_loop(0, PC, row_body, 0)
            copy_out(c, s).start()
            return carry

        lax.fori_loop(0, n_mine, chunk_body, 0)

        @pl.when(n_mine >= 2)
        def _():
            copy_out(n_mine - 2, lax.rem(n_mine - 2, 2)).wait()

        @pl.when(n_mine >= 1)
        def _():
            copy_out(n_mine - 1, lax.rem(n_mine - 1, 2)).wait()

    return sc_pack


def _make_sc_pool(B, L, V, D, num_cores, num_subcores):
    NW = num_cores * num_subcores
    assert B % NW == 0
    b_per_w = B // NW
    assert b_per_w % 2 == 0
    D32 = D // 32
    D2 = D // 2
    CH = 40
    assert L % CH == 0 and (CH * 4) % 8 == 0
    n_chunks = L // CH
    n_tot = b_per_w * n_chunks
    RU = 8
    assert CH % RU == 0

    mesh = plsc.VectorSubcoreMesh(core_axis_name="c", subcore_axis_name="s")

    @functools.partial(
        pl.kernel,
        mesh=mesh,
        out_type=jax.ShapeDtypeStruct((B, D), jnp.float32),
        scratch_types=[
            pltpu.VMEM((b_per_w * L,), jnp.int32),
            pltpu.VMEM((2, CH, D2), jnp.int32),
            pltpu.VMEM((2, D), jnp.float32),
            pltpu.SemaphoreType.DMA((2,)),
            pltpu.SemaphoreType.DMA((2,)),
        ],
        compiler_params=pltpu.CompilerParams(needs_layout_passes=False),
    )
    def sc_pool(tok_hbm, table_hbm, out_hbm, tok_v, rows_v, stage_v, gsem, osem):
        wid = lax.axis_index("s") * num_cores + lax.axis_index("c")
        base = wid * b_per_w
        pltpu.sync_copy(tok_hbm.at[pl.ds(base * L, b_per_w * L)], tok_v)

        def gather(k, s):
            return pltpu.make_async_copy(
                table_hbm.at[tok_v.at[pl.ds(k * CH, CH)]],
                rows_v.at[s],
                gsem.at[s],
            )

        gather(0, 0).start()

        def batch_body(i, carry):
            q = lax.rem(i, 2)

            def chunk_body(c, accs):
                k = i * n_chunks + c
                s = lax.rem(k, 2)
                gather(k, s).wait()

                @pl.when(k + 1 < n_tot)
                def _():
                    gather(k + 1, 1 - s).start()

                def row_body(r8, accs):
                    accs = list(accs)
                    for u in range(RU):
                        r = r8 * RU + u
                        for j in range(D32):
                            xi = rows_v[s, r, pl.ds(j * _LANES, _LANES)]
                            f_lo = plsc.bitcast(
                                lax.shift_left(xi, 16), jnp.float32
                            )
                            f_hi = plsc.bitcast(xi, jnp.float32)
                            accs[2 * j] = accs[2 * j] + f_lo
                            accs[2 * j + 1] = accs[2 * j + 1] + f_hi
                    return tuple(accs)

                return lax.fori_loop(0, CH // RU, row_body, accs)

            zeros = tuple(
                jnp.zeros((_LANES,), jnp.float32) for _ in range(2 * D32)
            )
            accs = lax.fori_loop(0, n_chunks, chunk_body, zeros)

            @pl.when(i >= 2)
            def _():
                pltpu.make_async_copy(
                    stage_v.at[q], out_hbm.at[base + i - 2], osem.at[q]
                ).wait()
            for j in range(D32):
                stage_v[q, pl.ds(j * 32, _LANES)] = accs[2 * j]
                stage_v[q, pl.ds(j * 32 + _LANES, _LANES)] = accs[2 * j + 1]
            pltpu.make_async_copy(
                stage_v.at[q], out_hbm.at[base + i], osem.at[q]
            ).start()
            return carry

        lax.fori_loop(0, b_per_w, batch_body, 0)

        pltpu.make_async_copy(
            stage_v.at[0], out_hbm.at[base + b_per_w - 2], osem.at[0]
        ).wait()
        pltpu.make_async_copy(
            stage_v.at[1], out_hbm.at[base + b_per_w - 1], osem.at[1]
        ).wait()

    return sc_pool


def _tc_proj_body(x_ref, w_ref, b_ref, o_ref, *, inv_l):
    x = x_ref[...] * inv_l
    y = lax.dot_general(
        x, w_ref[...], (((1,), (1,)), ((), ())),
        preferred_element_type=jnp.float32,
    )
    y = y + b_ref[...]
    norm = jnp.sqrt(jnp.sum(y * y, axis=-1, keepdims=True))
    o_ref[...] = y / jnp.maximum(norm, 1e-12)


def _tc_proj(x, W, b2d, L):
    B, D = x.shape
    BB = 256
    assert B % BB == 0
    return pl.pallas_call(
        functools.partial(_tc_proj_body, inv_l=1.0 / L),
        grid=(B // BB,),
        in_specs=[
            pl.BlockSpec((BB, D), lambda i: (i, 0)),
            pl.BlockSpec((D, D), lambda i: (0, 0)),
            pl.BlockSpec((1, D), lambda i: (0, 0)),
        ],
        out_specs=pl.BlockSpec((BB, D), lambda i: (i, 0)),
        out_shape=jax.ShapeDtypeStruct((B, D), jnp.float32),
    )(x, W, b2d)


def kernel(tokens, emb, W, b):
    B, L = tokens.shape
    V, D = emb.shape
    info = plsc.get_sparse_core_info()
    sc_pack = _make_sc_pack(V, D, info.num_cores, info.num_subcores)
    sc_pool = _make_sc_pool(B, L, V, D, info.num_cores, info.num_subcores)
    emb_bits = lax.bitcast_convert_type(emb, jnp.int32).reshape(V * D)
    packed = sc_pack(emb_bits).reshape(V, D // 2)
    pooled = sc_pool(tokens.astype(jnp.int32).reshape(B * L), packed)
    return _tc_proj(pooled, W, b.reshape(1, D), L)


# speedup vs baseline: 1.4867x; 1.3193x over previous
import functools

import jax
import jax.numpy as jnp
from jax import lax
from jax.experimental import pallas as pl
from jax.experimental.pallas import tpu as pltpu
from jax.experimental.pallas import tpu_sc as plsc

_LANES = 16


def _tc_pack_body(x_ref, o_ref):
    xi = lax.bitcast_convert_type(x_ref[...], jnp.int32)
    d2 = xi.shape[-1] // 2
    lo = lax.shift_right_logical(xi[:, :d2], 16)
    hi = jnp.bitwise_and(xi[:, d2:], jnp.int32(-65536))
    o_ref[...] = jnp.bitwise_or(lo, hi)


def _tc_pack(emb):
    V, D = emb.shape
    BB = 1024
    grid = (-(-V // BB),)
    return pl.pallas_call(
        _tc_pack_body,
        grid=grid,
        in_specs=[pl.BlockSpec((BB, D), lambda i: (i, 0))],
        out_specs=pl.BlockSpec((BB, D // 2), lambda i: (i, 0)),
        out_shape=jax.ShapeDtypeStruct((V, D // 2), jnp.int32),
    )(emb)


def _make_sc_pool(B, L, V, D, num_cores, num_subcores):
    NW = num_cores * num_subcores
    assert B % NW == 0
    b_per_w = B // NW
    assert b_per_w % 2 == 0
    D2 = D // 2
    DV = D2 // _LANES
    CH = 40
    assert L % CH == 0 and (CH * 4) % 8 == 0
    n_chunks = L // CH
    n_tot = b_per_w * n_chunks
    RU = 20
    assert CH % RU == 0

    mesh = plsc.VectorSubcoreMesh(core_axis_name="c", subcore_axis_name="s")

    @functools.partial(
        pl.kernel,
        mesh=mesh,
        out_type=jax.ShapeDtypeStruct((B, D), jnp.float32),
        scratch_types=[
            pltpu.VMEM((b_per_w * L,), jnp.int32),
            pltpu.VMEM((2, CH, D2), jnp.int32),
            pltpu.VMEM((2, D), jnp.float32),
            pltpu.SemaphoreType.DMA((2,)),
            pltpu.SemaphoreType.DMA((2,)),
        ],
        compiler_params=pltpu.CompilerParams(needs_layout_passes=False),
    )
    def sc_pool(tok_hbm, table_hbm, out_hbm, tok_v, rows_v, stage_v, gsem, osem):
        wid = lax.axis_index("s") * num_cores + lax.axis_index("c")
        base = wid * b_per_w
        pltpu.sync_copy(tok_hbm.at[pl.ds(base * L, b_per_w * L)], tok_v)

        def gather(k, s):
            return pltpu.make_async_copy(
                table_hbm.at[tok_v.at[pl.ds(k * CH, CH)]],
                rows_v.at[s],
                gsem.at[s],
            )

        gather(0, 0).start()

        def batch_body(i, carry):
            q = lax.rem(i, 2)

            def chunk_body(c, accs):
                k = i * n_chunks + c
                s = lax.rem(k, 2)
                gather(k, s).wait()

                @pl.when(k + 1 < n_tot)
                def _():
                    gather(k + 1, 1 - s).start()

                def row_body(rr, accs):
                    accs = list(accs)
                    for u in range(RU):
                        r = rr * RU + u
                        for j in range(DV):
                            xi = rows_v[s, r, pl.ds(j * _LANES, _LANES)]
                            f_lo = plsc.bitcast(
                                lax.shift_left(xi, 16), jnp.float32
                            )
                            f_hi = plsc.bitcast(xi, jnp.float32)
                            accs[j] = accs[j] + f_lo
                            accs[DV + j] = accs[DV + j] + f_hi
                    return tuple(accs)

                return lax.fori_loop(0, CH // RU, row_body, accs)

            zeros = tuple(
                jnp.zeros((_LANES,), jnp.float32) for _ in range(2 * DV)
            )
            accs = lax.fori_loop(0, n_chunks, chunk_body, zeros)

            @pl.when(i >= 2)
            def _():
                pltpu.make_async_copy(
                    stage_v.at[q], out_hbm.at[base + i - 2], osem.at[q]
                ).wait()
            for j in range(DV):
                stage_v[q, pl.ds(j * _LANES, _LANES)] = accs[j]
                stage_v[q, pl.ds(D2 + j * _LANES, _LANES)] = accs[DV + j]
            pltpu.make_async_copy(
                stage_v.at[q], out_hbm.at[base + i], osem.at[q]
            ).start()
            return carry

        lax.fori_loop(0, b_per_w, batch_body, 0)

        pltpu.make_async_copy(
            stage_v.at[0], out_hbm.at[base + b_per_w - 2], osem.at[0]
        ).wait()
        pltpu.make_async_copy(
            stage_v.at[1], out_hbm.at[base + b_per_w - 1], osem.at[1]
        ).wait()

    return sc_pool


def _tc_proj_body(x_ref, w_ref, b_ref, o_ref, *, inv_l):
    x = x_ref[...] * inv_l
    y = lax.dot_general(
        x, w_ref[...], (((1,), (1,)), ((), ())),
        preferred_element_type=jnp.float32,
    )
    y = y + b_ref[...]
    norm = jnp.sqrt(jnp.sum(y * y, axis=-1, keepdims=True))
    o_ref[...] = y / jnp.maximum(norm, 1e-12)


def _tc_proj(x, W, b2d, L):
    B, D = x.shape
    BB = 256
    assert B % BB == 0
    return pl.pallas_call(
        functools.partial(_tc_proj_body, inv_l=1.0 / L),
        grid=(B // BB,),
        in_specs=[
            pl.BlockSpec((BB, D), lambda i: (i, 0)),
            pl.BlockSpec((D, D), lambda i: (0, 0)),
            pl.BlockSpec((1, D), lambda i: (0, 0)),
        ],
        out_specs=pl.BlockSpec((BB, D), lambda i: (i, 0)),
        out_shape=jax.ShapeDtypeStruct((B, D), jnp.float32),
    )(x, W, b2d)


def kernel(tokens, emb, W, b):
    B, L = tokens.shape
    V, D = emb.shape
    info = plsc.get_sparse_core_info()
    sc_pool = _make_sc_pool(B, L, V, D, info.num_cores, info.num_subcores)
    packed = _tc_pack(emb)
    pooled = sc_pool(tokens.astype(jnp.int32).reshape(B * L), packed)
    return _tc_proj(pooled, W, b.reshape(1, D), L)


# speedup vs baseline: 2.7551x; 1.8531x over previous
import functools

import jax
import jax.numpy as jnp
from jax import lax
from jax.experimental import pallas as pl
from jax.experimental.pallas import tpu as pltpu
from jax.experimental.pallas import tpu_sc as plsc

_LANES = 16


def _tc_pack_body(x_ref, o_ref):
    xi = lax.bitcast_convert_type(x_ref[...], jnp.int32)
    d2 = xi.shape[-1] // 2
    lo = lax.shift_right_logical(xi[:, :d2], 16)
    hi = jnp.bitwise_and(xi[:, d2:], jnp.int32(-65536))
    o_ref[...] = jnp.bitwise_or(lo, hi)


def _tc_pack(emb):
    V, D = emb.shape
    BB = 1024
    grid = (-(-V // BB),)
    return pl.pallas_call(
        _tc_pack_body,
        grid=grid,
        in_specs=[pl.BlockSpec((BB, D), lambda i: (i, 0))],
        out_specs=pl.BlockSpec((BB, D // 2), lambda i: (i, 0)),
        out_shape=jax.ShapeDtypeStruct((V, D // 2), jnp.int32),
    )(emb)


def _make_sc_pool(B, L, V, D, num_cores, num_subcores):
    NW = num_cores * num_subcores
    assert B % NW == 0
    b_per_w = B // NW
    assert b_per_w % 2 == 0
    D2 = D // 2
    DV = D2 // _LANES
    CH = 40
    assert L % CH == 0 and (CH * 4) % 8 == 0
    n_chunks = L // CH
    n_tot = b_per_w * n_chunks
    RU = 20
    assert CH % RU == 0

    mesh = plsc.VectorSubcoreMesh(core_axis_name="c", subcore_axis_name="s")

    @functools.partial(
        pl.kernel,
        mesh=mesh,
        out_type=jax.ShapeDtypeStruct((B, D), jnp.float32),
        scratch_types=[
            pltpu.VMEM((b_per_w * L,), jnp.int32),
            pltpu.VMEM((4, CH, D2), jnp.int32),
            pltpu.VMEM((2, D), jnp.float32),
            pltpu.SemaphoreType.DMA((4,)),
            pltpu.SemaphoreType.DMA((2,)),
        ],
        compiler_params=pltpu.CompilerParams(needs_layout_passes=False),
    )
    def sc_pool(tok_hbm, table_hbm, out_hbm, tok_v, rows_v, stage_v, gsem, osem):
        wid = lax.axis_index("s") * num_cores + lax.axis_index("c")
        base = wid * b_per_w
        pltpu.sync_copy(tok_hbm.at[pl.ds(base * L, b_per_w * L)], tok_v)

        def gather(k, s):
            return pltpu.make_async_copy(
                table_hbm.at[tok_v.at[pl.ds(k * CH, CH)]],
                rows_v.at[s],
                gsem.at[s],
            )

        gather(0, 0).start()
        gather(1, 1).start()
        gather(2, 2).start()

        def batch_body(i, carry):
            q = lax.rem(i, 2)

            def chunk_body(c, accs):
                k = i * n_chunks + c
                s = lax.rem(k, 4)
                gather(k, s).wait()

                @pl.when(k + 3 < n_tot)
                def _():
                    gather(k + 3, lax.rem(k + 3, 4)).start()

                def row_body(rr, accs):
                    accs = list(accs)
                    for u in range(RU):
                        r = rr * RU + u
                        for j in range(DV):
                            xi = rows_v[s, r, pl.ds(j * _LANES, _LANES)]
                            f_lo = plsc.bitcast(
                                lax.shift_left(xi, 16), jnp.float32
                            )
                            f_hi = plsc.bitcast(xi, jnp.float32)
                            accs[j] = accs[j] + f_lo
                            accs[DV + j] = accs[DV + j] + f_hi
                    return tuple(accs)

                return lax.fori_loop(0, CH // RU, row_body, accs)

            zeros = tuple(
                jnp.zeros((_LANES,), jnp.float32) for _ in range(2 * DV)
            )
            accs = lax.fori_loop(0, n_chunks, chunk_body, zeros)

            @pl.when(i >= 2)
            def _():
                pltpu.make_async_copy(
                    stage_v.at[q], out_hbm.at[base + i - 2], osem.at[q]
                ).wait()
            for j in range(DV):
                stage_v[q, pl.ds(j * _LANES, _LANES)] = accs[j]
                stage_v[q, pl.ds(D2 + j * _LANES, _LANES)] = accs[DV + j]
            pltpu.make_async_copy(
                stage_v.at[q], out_hbm.at[base + i], osem.at[q]
            ).start()
            return carry

        lax.fori_loop(0, b_per_w, batch_body, 0)

        pltpu.make_async_copy(
            stage_v.at[0], out_hbm.at[base + b_per_w - 2], osem.at[0]
        ).wait()
        pltpu.make_async_copy(
            stage_v.at[1], out_hbm.at[base + b_per_w - 1], osem.at[1]
        ).wait()

    return sc_pool


def _tc_proj_body(x_ref, w_ref, b_ref, o_ref, *, inv_l):
    x = x_ref[...] * inv_l
    y = lax.dot_general(
        x, w_ref[...], (((1,), (1,)), ((), ())),
        preferred_element_type=jnp.float32,
    )
    y = y + b_ref[...]
    norm = jnp.sqrt(jnp.sum(y * y, axis=-1, keepdims=True))
    o_ref[...] = y / jnp.maximum(norm, 1e-12)


def _tc_proj(x, W, b2d, L):
    B, D = x.shape
    BB = 256
    assert B % BB == 0
    return pl.pallas_call(
        functools.partial(_tc_proj_body, inv_l=1.0 / L),
        grid=(B // BB,),
        in_specs=[
            pl.BlockSpec((BB, D), lambda i: (i, 0)),
            pl.BlockSpec((D, D), lambda i: (0, 0)),
            pl.BlockSpec((1, D), lambda i: (0, 0)),
        ],
        out_specs=pl.BlockSpec((BB, D), lambda i: (i, 0)),
        out_shape=jax.ShapeDtypeStruct((B, D), jnp.float32),
    )(x, W, b2d)


def kernel(tokens, emb, W, b):
    B, L = tokens.shape
    V, D = emb.shape
    info = plsc.get_sparse_core_info()
    sc_pool = _make_sc_pool(B, L, V, D, info.num_cores, info.num_subcores)
    packed = _tc_pack(emb)
    pooled = sc_pool(tokens.astype(jnp.int32).reshape(B * L), packed)
    return _tc_proj(pooled, W, b.reshape(1, D), L)


# speedup vs baseline: 2.7933x; 1.0139x over previous
import functools

import jax
import jax.numpy as jnp
from jax import lax
from jax.experimental import pallas as pl
from jax.experimental.pallas import tpu as pltpu
from jax.experimental.pallas import tpu_sc as plsc

_LANES = 16


def _tc_pack_body(x_ref, o_ref):
    xi = lax.bitcast_convert_type(x_ref[...], jnp.int32)
    d2 = xi.shape[-1] // 2
    lo = lax.shift_right_logical(xi[:, :d2], 16)
    hi = jnp.bitwise_and(xi[:, d2:], jnp.int32(-65536))
    o_ref[...] = jnp.bitwise_or(lo, hi)


def _tc_pack(emb):
    V, D = emb.shape
    BB = 1024
    grid = (-(-V // BB),)
    return pl.pallas_call(
        _tc_pack_body,
        grid=grid,
        in_specs=[pl.BlockSpec((BB, D), lambda i: (i, 0))],
        out_specs=pl.BlockSpec((BB, D // 2), lambda i: (i, 0)),
        out_shape=jax.ShapeDtypeStruct((V, D // 2), jnp.int32),
    )(emb)


def _make_sc_pool(B, L, V, D, num_cores, num_subcores):
    NW = num_cores * num_subcores
    assert B % NW == 0
    b_per_w = B // NW
    assert b_per_w % 2 == 0
    D2 = D // 2
    DV = D2 // _LANES
    CH = 40
    assert L % CH == 0 and (CH * 4) % 8 == 0
    n_chunks = L // CH
    n_tot = b_per_w * n_chunks
    RU = 20
    assert CH % RU == 0

    mesh = plsc.VectorSubcoreMesh(core_axis_name="c", subcore_axis_name="s")

    @functools.partial(
        pl.kernel,
        mesh=mesh,
        out_type=jax.ShapeDtypeStruct((B, D), jnp.float32),
        scratch_types=[
            pltpu.VMEM((b_per_w * L,), jnp.int32),
            pltpu.VMEM((6, CH, D2), jnp.int32),
            pltpu.VMEM((2, D), jnp.float32),
            pltpu.SemaphoreType.DMA((6,)),
            pltpu.SemaphoreType.DMA((2,)),
        ],
        compiler_params=pltpu.CompilerParams(needs_layout_passes=False),
    )
    def sc_pool(tok_hbm, table_hbm, out_hbm, tok_v, rows_v, stage_v, gsem, osem):
        wid = lax.axis_index("s") * num_cores + lax.axis_index("c")
        base = wid * b_per_w
        pltpu.sync_copy(tok_hbm.at[pl.ds(base * L, b_per_w * L)], tok_v)

        def gather(k, s):
            return pltpu.make_async_copy(
                table_hbm.at[tok_v.at[pl.ds(k * CH, CH)]],
                rows_v.at[s],
                gsem.at[s],
            )

        for p in range(5):
            gather(p, p).start()

        def batch_body(i, carry):
            q = lax.rem(i, 2)

            def chunk_body(c, accs):
                k = i * n_chunks + c
                s = lax.rem(k, 6)
                gather(k, s).wait()

                @pl.when(k + 5 < n_tot)
                def _():
                    gather(k + 5, lax.rem(k + 5, 6)).start()

                def row_body(rr, accs):
                    accs = list(accs)
                    for u in range(RU):
                        r = rr * RU + u
                        for j in range(DV):
                            xi = rows_v[s, r, pl.ds(j * _LANES, _LANES)]
                            f_lo = plsc.bitcast(
                                lax.shift_left(xi, 16), jnp.float32
                            )
                            f_hi = plsc.bitcast(xi, jnp.float32)
                            accs[j] = accs[j] + f_lo
                            accs[DV + j] = accs[DV + j] + f_hi
                    return tuple(accs)

                return lax.fori_loop(0, CH // RU, row_body, accs)

            zeros = tuple(
                jnp.zeros((_LANES,), jnp.float32) for _ in range(2 * DV)
            )
            accs = lax.fori_loop(0, n_chunks, chunk_body, zeros)

            @pl.when(i >= 2)
            def _():
                pltpu.make_async_copy(
                    stage_v.at[q], out_hbm.at[base + i - 2], osem.at[q]
                ).wait()
            for j in range(DV):
                stage_v[q, pl.ds(j * _LANES, _LANES)] = accs[j]
                stage_v[q, pl.ds(D2 + j * _LANES, _LANES)] = accs[DV + j]
            pltpu.make_async_copy(
                stage_v.at[q], out_hbm.at[base + i], osem.at[q]
            ).start()
            return carry

        lax.fori_loop(0, b_per_w, batch_body, 0)

        pltpu.make_async_copy(
            stage_v.at[0], out_hbm.at[base + b_per_w - 2], osem.at[0]
        ).wait()
        pltpu.make_async_copy(
            stage_v.at[1], out_hbm.at[base + b_per_w - 1], osem.at[1]
        ).wait()

    return sc_pool


def _tc_proj_body(x_ref, w_ref, b_ref, o_ref, *, inv_l):
    x = x_ref[...] * inv_l
    y = lax.dot_general(
        x, w_ref[...], (((1,), (1,)), ((), ())),
        preferred_element_type=jnp.float32,
    )
    y = y + b_ref[...]
    norm = jnp.sqrt(jnp.sum(y * y, axis=-1, keepdims=True))
    o_ref[...] = y / jnp.maximum(norm, 1e-12)


def _tc_proj(x, W, b2d, L):
    B, D = x.shape
    BB = 256
    assert B % BB == 0
    return pl.pallas_call(
        functools.partial(_tc_proj_body, inv_l=1.0 / L),
        grid=(B // BB,),
        in_specs=[
            pl.BlockSpec((BB, D), lambda i: (i, 0)),
            pl.BlockSpec((D, D), lambda i: (0, 0)),
            pl.BlockSpec((1, D), lambda i: (0, 0)),
        ],
        out_specs=pl.BlockSpec((BB, D), lambda i: (i, 0)),
        out_shape=jax.ShapeDtypeStruct((B, D), jnp.float32),
    )(x, W, b2d)


def kernel(tokens, emb, W, b):
    B, L = tokens.shape
    V, D = emb.shape
    info = plsc.get_sparse_core_info()
    sc_pool = _make_sc_pool(B, L, V, D, info.num_cores, info.num_subcores)
    packed = _tc_pack(emb)
    pooled = sc_pool(tokens.astype(jnp.int32).reshape(B * L), packed)
    return _tc_proj(pooled, W, b.reshape(1, D), L)


# speedup vs baseline: 2.8255x; 1.0115x over previous
import functools

import jax
import jax.numpy as jnp
from jax import lax
from jax.experimental import pallas as pl
from jax.experimental.pallas import tpu as pltpu
from jax.experimental.pallas import tpu_sc as plsc

_LANES = 16


def _tc_pack_body(x_ref, o_ref):
    xi = lax.bitcast_convert_type(x_ref[...], jnp.int32)
    d2 = xi.shape[-1] // 2
    lo = lax.shift_right_logical(xi[:, :d2], 16)
    hi = jnp.bitwise_and(xi[:, d2:], jnp.int32(-65536))
    o_ref[...] = jnp.bitwise_or(lo, hi)


def _tc_pack(emb):
    V, D = emb.shape
    BB = 1024
    grid = (-(-V // BB),)
    return pl.pallas_call(
        _tc_pack_body,
        grid=grid,
        in_specs=[pl.BlockSpec((BB, D), lambda i: (i, 0))],
        out_specs=pl.BlockSpec((BB, D // 2), lambda i: (i, 0)),
        out_shape=jax.ShapeDtypeStruct((V, D // 2), jnp.int32),
    )(emb)


def _make_sc_pool(B, L, V, D, num_cores, num_subcores):
    NW = num_cores * num_subcores
    assert B % NW == 0
    b_per_w = B // NW
    assert b_per_w % 2 == 0
    D2 = D // 2
    DV = D2 // _LANES
    CH = 40
    assert L % CH == 0 and (CH * 4) % 8 == 0
    n_chunks = L // CH
    n_tot = b_per_w * n_chunks
    RU = 20
    assert CH % RU == 0

    mesh = plsc.VectorSubcoreMesh(core_axis_name="c", subcore_axis_name="s")

    @functools.partial(
        pl.kernel,
        mesh=mesh,
        out_type=jax.ShapeDtypeStruct((B, D), jnp.float32),
        scratch_types=[
            pltpu.VMEM((b_per_w * L,), jnp.int32),
            pltpu.VMEM((6, CH, D2), jnp.int32),
            pltpu.VMEM((2, D), jnp.float32),
            pltpu.SemaphoreType.DMA((6,)),
            pltpu.SemaphoreType.DMA((2,)),
        ],
        compiler_params=pltpu.CompilerParams(needs_layout_passes=False),
    )
    def sc_pool(tok_hbm, table_hbm, out_hbm, tok_v, rows_v, stage_v, gsem, osem):
        wid = lax.axis_index("s") * num_cores + lax.axis_index("c")
        base = wid * b_per_w
        pltpu.sync_copy(tok_hbm.at[pl.ds(base * L, b_per_w * L)], tok_v)

        def gather(k, s):
            return pltpu.make_async_copy(
                table_hbm.at[tok_v.at[pl.ds(k * CH, CH)]],
                rows_v.at[s],
                gsem.at[s],
            )

        for p in range(5):
            gather(p, p).start()

        def batch_body(i, carry):
            q = lax.rem(i, 2)

            def chunk_body(c, accs):
                k = i * n_chunks + c
                s = lax.rem(k, 6)
                gather(k, s).wait()

                @pl.when(k + 5 < n_tot)
                def _():
                    gather(k + 5, lax.rem(k + 5, 6)).start()

                def row_body(rr, accs):
                    accs = list(accs)
                    for u in range(RU):
                        r = rr * RU + u
                        for j in range(DV):
                            xi = rows_v[s, r, pl.ds(j * _LANES, _LANES)]
                            f_lo = plsc.bitcast(
                                lax.shift_left(xi, 16), jnp.float32
                            )
                            f_hi = plsc.bitcast(xi, jnp.float32)
                            accs[j] = accs[j] + f_lo
                            accs[DV + j] = accs[DV + j] + f_hi
                    return tuple(accs)

                return lax.fori_loop(0, CH // RU, row_body, accs)

            zeros = tuple(
                jnp.zeros((_LANES,), jnp.float32) for _ in range(2 * DV)
            )
            accs = lax.fori_loop(0, n_chunks, chunk_body, zeros)

            @pl.when(i >= 2)
            def _():
                pltpu.make_async_copy(
                    stage_v.at[q], out_hbm.at[base + i - 2], osem.at[q]
                ).wait()
            for j in range(DV):
                stage_v[q, pl.ds(j * _LANES, _LANES)] = accs[j]
                stage_v[q, pl.ds(D2 + j * _LANES, _LANES)] = accs[DV + j]
            pltpu.make_async_copy(
                stage_v.at[q], out_hbm.at[base + i], osem.at[q]
            ).start()
            return carry

        lax.fori_loop(0, b_per_w, batch_body, 0)

        pltpu.make_async_copy(
            stage_v.at[0], out_hbm.at[base + b_per_w - 2], osem.at[0]
        ).wait()
        pltpu.make_async_copy(
            stage_v.at[1], out_hbm.at[base + b_per_w - 1], osem.at[1]
        ).wait()

    return sc_pool


def _tc_proj_body(x_ref, w_ref, b_ref, o_ref, *, inv_l):
    x = (x_ref[...] * inv_l).astype(jnp.bfloat16)
    y = lax.dot_general(
        x, w_ref[...].astype(jnp.bfloat16), (((1,), (1,)), ((), ())),
        preferred_element_type=jnp.float32,
    )
    y = y + b_ref[...]
    norm = jnp.sqrt(jnp.sum(y * y, axis=-1, keepdims=True))
    o_ref[...] = y / jnp.maximum(norm, 1e-12)


def _tc_proj(x, W, b2d, L):
    B, D = x.shape
    BB = 512
    assert B % BB == 0
    return pl.pallas_call(
        functools.partial(_tc_proj_body, inv_l=1.0 / L),
        grid=(B // BB,),
        in_specs=[
            pl.BlockSpec((BB, D), lambda i: (i, 0)),
            pl.BlockSpec((D, D), lambda i: (0, 0)),
            pl.BlockSpec((1, D), lambda i: (0, 0)),
        ],
        out_specs=pl.BlockSpec((BB, D), lambda i: (i, 0)),
        out_shape=jax.ShapeDtypeStruct((B, D), jnp.float32),
    )(x, W, b2d)


def kernel(tokens, emb, W, b):
    B, L = tokens.shape
    V, D = emb.shape
    info = plsc.get_sparse_core_info()
    sc_pool = _make_sc_pool(B, L, V, D, info.num_cores, info.num_subcores)
    packed = _tc_pack(emb)
    pooled = sc_pool(tokens.astype(jnp.int32).reshape(B * L), packed)
    return _tc_proj(pooled, W, b.reshape(1, D), L)
